# Initial kernel scaffold; baseline (speedup 1.0000x reference)
import jax
import jax.numpy as jnp
from jax.experimental import pallas as pl


def kernel(x):
    raise NotImplementedError("write your pallas kernel here")


# speedup vs baseline: 118.2101x; 118.2101x over previous
import functools

import jax
import jax.numpy as jnp
from jax import lax
from jax.experimental import pallas as pl
from jax.experimental.pallas import tpu as pltpu
from jax.experimental.pallas import tpu_sc as plsc

_M, _D = 16384, 128
_N = _M * _D // 2
_NC, _NS = 2, 16
_NW = _NC * _NS
_CHUNK = _N // _NW
_ROWS = _CHUNK * 2 // _D

_mesh = plsc.VectorSubcoreMesh(core_axis_name="c", subcore_axis_name="s")


@functools.partial(
    pl.kernel,
    out_type=jax.ShapeDtypeStruct((_M, _D), jnp.float32),
    mesh=_mesh,
    scratch_types=[
        pltpu.VMEM((_CHUNK // _D, _D), jnp.float32),
        pltpu.VMEM((_ROWS, _D), jnp.float32),
    ],
    compiler_params=pltpu.CompilerParams(needs_layout_passes=False),
)
def _extend(x_hbm, out_hbm, xbuf, buf):
    wid = lax.axis_index("s") * _NC + lax.axis_index("c")
    nrows = _CHUNK // _D
    xrow0 = wid * nrows
    row0 = wid * _ROWS

    pltpu.sync_copy(x_hbm.at[pl.ds(xrow0, nrows), :], xbuf)

    lanes = lax.iota(jnp.int32, 16)
    odd_cols = [lanes * 2 + 1 + 32 * t for t in range(4)]
    even_cols = [lanes * 2 + 32 * t for t in range(4)]
    nan16 = jnp.full((16,), jnp.nan, jnp.float32)
    one16 = jnp.full((16,), 1, jnp.int32)

    def body(i, row):
        for p in range(2):
            rowv = row + (p * one16 if p else 0)
            for t in range(4):
                v = xbuf[i, pl.ds(64 * p + 16 * t, 16)]
                plsc.store_scatter(buf, [rowv, odd_cols[t]], v)
                plsc.store_scatter(buf, [rowv, even_cols[t]], nan16)
        return row + 2 * one16

    lax.fori_loop(0, nrows, body, jnp.zeros((16,), jnp.int32))

    pltpu.sync_copy(buf, out_hbm.at[pl.ds(row0, _ROWS), :])


def kernel(x):
    return _extend(x)


# speedup vs baseline: 144.4540x; 1.2220x over previous
import functools

import jax
import jax.numpy as jnp
from jax import lax
from jax.experimental import pallas as pl
from jax.experimental.pallas import tpu as pltpu
from jax.experimental.pallas import tpu_sc as plsc

_M, _D = 16384, 128
_N = _M * _D // 2
_NC, _NS = 2, 16
_NW = _NC * _NS
_CHUNK = _N // _NW
_ROWS = _CHUNK * 2 // _D

_mesh = plsc.VectorSubcoreMesh(core_axis_name="c", subcore_axis_name="s")


@functools.partial(
    pl.kernel,
    out_type=jax.ShapeDtypeStruct((_M, _D), jnp.float32),
    mesh=_mesh,
    scratch_types=[
        pltpu.VMEM((_CHUNK // _D, _D), jnp.float32),
        pltpu.VMEM((_ROWS, _D), jnp.float32),
        pltpu.SemaphoreType.DMA,
    ],
    compiler_params=pltpu.CompilerParams(needs_layout_passes=False),
)
def _extend(x_hbm, out_hbm, xbuf, buf, sem):
    wid = lax.axis_index("s") * _NC + lax.axis_index("c")
    nrows = _CHUNK // _D
    xrow0 = wid * nrows
    row0 = wid * _ROWS

    in_copy = pltpu.make_async_copy(x_hbm.at[pl.ds(xrow0, nrows), :], xbuf,
                                    sem)
    in_copy.start()

    lanes = lax.iota(jnp.int32, 16)
    odd_cols = [lanes * 2 + 1 + 32 * t for t in range(4)]
    even_cols = [lanes * 2 + 32 * t for t in range(4)]
    nan16 = jnp.full((16,), jnp.nan, jnp.float32)
    one16 = jnp.full((16,), 1, jnp.int32)

    def nan_body(i, row):
        rows = [row, row + one16]
        for p in range(2):
            for t in range(4):
                plsc.store_scatter(buf, [rows[p], even_cols[t]], nan16)
        return row + 2 * one16

    lax.fori_loop(0, nrows, nan_body, jnp.zeros((16,), jnp.int32))

    in_copy.wait()

    def body(i, row):
        rows = [row, row + one16]
        vals = [xbuf[i, pl.ds(16 * t, 16)] for t in range(8)]
        for t in range(8):
            plsc.store_scatter(buf, [rows[t // 4], odd_cols[t % 4]], vals[t])
        return row + 2 * one16

    lax.fori_loop(0, nrows, body, jnp.zeros((16,), jnp.int32))

    pltpu.sync_copy(buf, out_hbm.at[pl.ds(row0, _ROWS), :])


def kernel(x):
    return _extend(x)
